# Initial kernel scaffold; baseline (speedup 1.0000x reference)
import jax
import jax.numpy as jnp
from jax.experimental import pallas as pl


def kernel(ct_w, ct_b, c1_w, c1_b, bn1_g, bn1_b, bn1_m, bn1_v, c2_w, c2_b, bn2_g, bn2_b, bn2_m, bn2_v, x_nchw, xcopy_nchw):
    raise NotImplementedError("write your pallas kernel here")


# speedup vs baseline: 1.4196x; 1.4196x over previous
import jax
import jax.numpy as jnp
from jax.experimental import pallas as pl
from jax.experimental.pallas import tpu as pltpu

_BN_EPS = 1e-5
_NSTRIPS = 2
_VMEM_LIMIT = 56 * 1024 * 1024


def _fused_kernel(x_hbm, xc_hbm, wd_ref, bd_ref, w1_ref, s1_ref, b1_ref,
                  w2_ref, s2_ref, b2_ref, o_ref, sx, scf, su, hb, sem):
    b = pl.program_id(0)
    r = pl.program_id(1)
    nr = pl.num_programs(1)
    _, th, wo, co = o_ref.shape
    wi, ci = x_hbm.shape[2], x_hbm.shape[3]
    hh = th // 2

    cp_x = pltpu.make_async_copy(
        x_hbm.at[b, pl.ds(r * hh, hh), :, :], sx.at[pl.ds(1, hh)],
        sem.at[0, 0])
    cp_x.start()
    cp_c = pltpu.make_async_copy(
        xc_hbm.at[b, pl.ds(r * th, th), :, :], scf.at[pl.ds(2, th)],
        sem.at[1, 0])
    cp_c.start()

    @pl.when(r > 0)
    def _():
        pltpu.make_async_copy(x_hbm.at[b, pl.ds(r * hh - 1, 1), :, :],
                              sx.at[pl.ds(0, 1)], sem.at[0, 1]).start()
        pltpu.make_async_copy(xc_hbm.at[b, pl.ds(r * th - 2, 2), :, :],
                              scf.at[pl.ds(0, 2)], sem.at[1, 1]).start()

    @pl.when(r < nr - 1)
    def _():
        pltpu.make_async_copy(x_hbm.at[b, pl.ds((r + 1) * hh, 1), :, :],
                              sx.at[pl.ds(hh + 1, 1)], sem.at[0, 2]).start()
        pltpu.make_async_copy(xc_hbm.at[b, pl.ds((r + 1) * th, 2), :, :],
                              scf.at[pl.ds(th + 2, 2)], sem.at[1, 2]).start()

    @pl.when(r == 0)
    def _():
        sx[0:1] = jnp.zeros((1, wi, ci), jnp.float32)
        scf[0:2] = jnp.zeros((2, wo, co), jnp.float32)

    @pl.when(r == nr - 1)
    def _():
        sx[hh + 1:hh + 2] = jnp.zeros((1, wi, ci), jnp.float32)
        scf[th + 2:th + 4] = jnp.zeros((2, wo, co), jnp.float32)

    cp_x.wait()

    @pl.when(r > 0)
    def _():
        pltpu.make_async_copy(x_hbm.at[b, pl.ds(r * hh - 1, 1), :, :],
                              sx.at[pl.ds(0, 1)], sem.at[0, 1]).wait()

    @pl.when(r < nr - 1)
    def _():
        pltpu.make_async_copy(x_hbm.at[b, pl.ds((r + 1) * hh, 1), :, :],
                              sx.at[pl.ds(hh + 1, 1)], sem.at[0, 2]).wait()

    x2 = sx[...].reshape((hh + 2) * wi, ci).astype(jnp.bfloat16)
    row_blocks = []
    for ky in range(2):
        parts = []
        for kx in range(2):
            y = jnp.dot(x2, wd_ref[ky, kx],
                        preferred_element_type=jnp.float32) + bd_ref[0]
            parts.append(y.astype(jnp.bfloat16).reshape(hh + 2, wi, 1, co))
        blk = jnp.concatenate(parts, axis=2)
        row_blocks.append(blk.reshape(hh + 2, 1, wo, co))
    up = jnp.concatenate(row_blocks, axis=1).reshape(th + 4, wo, co)
    su[:, 1:wo + 1, 0:co] = up

    cp_c.wait()

    @pl.when(r > 0)
    def _():
        pltpu.make_async_copy(xc_hbm.at[b, pl.ds(r * th - 2, 2), :, :],
                              scf.at[pl.ds(0, 2)], sem.at[1, 1]).wait()

    @pl.when(r < nr - 1)
    def _():
        pltpu.make_async_copy(xc_hbm.at[b, pl.ds((r + 1) * th, 2), :, :],
                              scf.at[pl.ds(th + 2, 2)], sem.at[1, 2]).wait()

    su[:, 1:wo + 1, co:2 * co] = scf[...].astype(jnp.bfloat16)

    su[:, 0:1, :] = jnp.zeros((th + 4, 1, 2 * co), jnp.bfloat16)
    su[:, wo + 1:wo + 2, :] = jnp.zeros((th + 4, 1, 2 * co), jnp.bfloat16)

    @pl.when(r == 0)
    def _():
        su[0:2] = jnp.zeros((2, wo + 2, 2 * co), jnp.bfloat16)

    @pl.when(r == nr - 1)
    def _():
        su[th + 2:th + 4] = jnp.zeros((2, wo + 2, 2 * co), jnp.bfloat16)

    m1 = (th + 2) * wo
    acc = jnp.zeros((m1, co), jnp.float32)
    for dx in range(3):
        q = su[:, dx:dx + wo, :].reshape((th + 4) * wo, 2 * co)
        for dy in range(3):
            acc = acc + jnp.dot(q[dy * wo:dy * wo + m1], w1_ref[dy, dx],
                                preferred_element_type=jnp.float32)
    y1 = jnp.maximum(acc * s1_ref[0] + b1_ref[0], 0.0)
    hb[:, 1:wo + 1, :] = y1.astype(jnp.bfloat16).reshape(th + 2, wo, co)
    hb[:, 0:1, :] = jnp.zeros((th + 2, 1, co), jnp.bfloat16)
    hb[:, wo + 1:wo + 2, :] = jnp.zeros((th + 2, 1, co), jnp.bfloat16)

    @pl.when(r == 0)
    def _():
        hb[0:1] = jnp.zeros((1, wo + 2, co), jnp.bfloat16)

    @pl.when(r == nr - 1)
    def _():
        hb[th + 1:th + 2] = jnp.zeros((1, wo + 2, co), jnp.bfloat16)

    m2 = th * wo
    acc2 = jnp.zeros((m2, co), jnp.float32)
    for dx in range(3):
        q = hb[:, dx:dx + wo, :].reshape((th + 2) * wo, co)
        for dy in range(3):
            acc2 = acc2 + jnp.dot(q[dy * wo:dy * wo + m2], w2_ref[dy, dx],
                                  preferred_element_type=jnp.float32)
    y2 = jnp.maximum(acc2 * s2_ref[0] + b2_ref[0], 0.0)
    o_ref[0] = y2.reshape(th, wo, co)


def _fold_bn(conv_b, gamma, beta, mean, var):
    scale = gamma / jnp.sqrt(var + _BN_EPS)
    bias = beta + (conv_b - mean) * scale
    return scale[None, :], bias[None, :]


@jax.jit
def _forward(ct_w, ct_b, c1_w, c1_b, bn1_g, bn1_b, bn1_m, bn1_v,
             c2_w, c2_b, bn2_g, bn2_b, bn2_m, bn2_v, x_nchw, xcopy_nchw):
    n, ci, hi, wi = x_nchw.shape
    co = ct_b.shape[0]
    ho, wo = 2 * hi, 2 * wi
    th = ho // _NSTRIPS

    x = jnp.transpose(x_nchw, (0, 2, 3, 1))
    xc = jnp.transpose(xcopy_nchw, (0, 2, 3, 1))

    wd = jnp.transpose(ct_w, (2, 3, 0, 1)).astype(jnp.bfloat16)
    bd = ct_b[None, :]
    w1 = jnp.transpose(c1_w, (2, 3, 1, 0)).astype(jnp.bfloat16)
    s1, b1 = _fold_bn(c1_b, bn1_g, bn1_b, bn1_m, bn1_v)
    w2 = jnp.transpose(c2_w, (2, 3, 1, 0)).astype(jnp.bfloat16)
    s2, b2 = _fold_bn(c2_b, bn2_g, bn2_b, bn2_m, bn2_v)

    out = pl.pallas_call(
        _fused_kernel,
        grid=(n, _NSTRIPS),
        in_specs=[
            pl.BlockSpec(memory_space=pl.ANY),
            pl.BlockSpec(memory_space=pl.ANY),
            pl.BlockSpec((2, 2, ci, co), lambda b, r: (0, 0, 0, 0)),
            pl.BlockSpec((1, co), lambda b, r: (0, 0)),
            pl.BlockSpec((3, 3, 2 * co, co), lambda b, r: (0, 0, 0, 0)),
            pl.BlockSpec((1, co), lambda b, r: (0, 0)),
            pl.BlockSpec((1, co), lambda b, r: (0, 0)),
            pl.BlockSpec((3, 3, co, co), lambda b, r: (0, 0, 0, 0)),
            pl.BlockSpec((1, co), lambda b, r: (0, 0)),
            pl.BlockSpec((1, co), lambda b, r: (0, 0)),
        ],
        out_specs=pl.BlockSpec((1, th, wo, co), lambda b, r: (b, r, 0, 0)),
        out_shape=jax.ShapeDtypeStruct((n, ho, wo, co), jnp.float32),
        scratch_shapes=[
            pltpu.VMEM((th // 2 + 2, wi, ci), jnp.float32),
            pltpu.VMEM((th + 4, wo, co), jnp.float32),
            pltpu.VMEM((th + 4, wo + 2, 2 * co), jnp.bfloat16),
            pltpu.VMEM((th + 2, wo + 2, co), jnp.bfloat16),
            pltpu.SemaphoreType.DMA((2, 3)),
        ],
        compiler_params=pltpu.CompilerParams(
            dimension_semantics=("parallel", "parallel"),
            vmem_limit_bytes=_VMEM_LIMIT),
    )(x, xc, wd, bd, w1, s1, b1, w2, s2, b2)
    return jnp.transpose(out, (0, 3, 1, 2))


def kernel(ct_w, ct_b, c1_w, c1_b, bn1_g, bn1_b, bn1_m, bn1_v,
           c2_w, c2_b, bn2_g, bn2_b, bn2_m, bn2_v, x_nchw, xcopy_nchw):
    return _forward(ct_w, ct_b, c1_w, c1_b, bn1_g, bn1_b, bn1_m, bn1_v,
                    c2_w, c2_b, bn2_g, bn2_b, bn2_m, bn2_v,
                    x_nchw, xcopy_nchw)


# speedup vs baseline: 1.9984x; 1.4077x over previous
import jax
import jax.numpy as jnp
from jax.experimental import pallas as pl
from jax.experimental.pallas import tpu as pltpu

_BN_EPS = 1e-5
_NSTRIPS = 2
_VMEM_LIMIT = 60 * 1024 * 1024


def _shift_cols(buf, d, out):
    rows, w, c = buf.shape
    zcol = jnp.zeros((rows, 1, c), buf.dtype)
    if d == -1:
        out[:, 1:w, :] = buf[:, 0:w - 1, :]
        out[:, 0:1, :] = zcol
    else:
        out[:, 0:w - 1, :] = buf[:, 1:w, :]
        out[:, w - 1:w, :] = zcol


def _conv3x3(bufs, wm_ref, m_out, w_stride):
    acc = None
    co3 = wm_ref.shape[2]
    co = co3 // 3
    for dx in range(3):
        rows, w, c = bufs[dx].shape
        q = bufs[dx][...].reshape(rows * w, c)
        y = jnp.dot(q, wm_ref[dx], preferred_element_type=jnp.float32)
        for dy in range(3):
            t = y[dy * w_stride:dy * w_stride + m_out, dy * co:(dy + 1) * co]
            acc = t if acc is None else acc + t
    return acc


def _fused_kernel(x_hbm, xc_hbm, wd_ref, bd_ref, w1_ref, s1_ref, b1_ref,
                  w2_ref, s2_ref, b2_ref, o_ref,
                  sx, scf, su, sl, sr, hb, hl, hr, sem):
    b = pl.program_id(0)
    r = pl.program_id(1)
    nr = pl.num_programs(1)
    _, th, wo, co = o_ref.shape
    wi, ci = x_hbm.shape[2], x_hbm.shape[3]
    hh = th // 2

    cp_x = pltpu.make_async_copy(
        x_hbm.at[b, pl.ds(r * hh, hh), :, :], sx.at[pl.ds(1, hh)],
        sem.at[0, 0])
    cp_x.start()
    cp_c = pltpu.make_async_copy(
        xc_hbm.at[b, pl.ds(r * th, th), :, :], scf.at[pl.ds(2, th)],
        sem.at[1, 0])
    cp_c.start()

    @pl.when(r > 0)
    def _():
        pltpu.make_async_copy(x_hbm.at[b, pl.ds(r * hh - 1, 1), :, :],
                              sx.at[pl.ds(0, 1)], sem.at[0, 1]).start()
        pltpu.make_async_copy(xc_hbm.at[b, pl.ds(r * th - 2, 2), :, :],
                              scf.at[pl.ds(0, 2)], sem.at[1, 1]).start()

    @pl.when(r < nr - 1)
    def _():
        pltpu.make_async_copy(x_hbm.at[b, pl.ds((r + 1) * hh, 1), :, :],
                              sx.at[pl.ds(hh + 1, 1)], sem.at[0, 2]).start()
        pltpu.make_async_copy(xc_hbm.at[b, pl.ds((r + 1) * th, 2), :, :],
                              scf.at[pl.ds(th + 2, 2)], sem.at[1, 2]).start()

    @pl.when(r == 0)
    def _():
        sx[0:1] = jnp.zeros((1, wi, ci), jnp.float32)

    @pl.when(r == nr - 1)
    def _():
        sx[hh + 1:hh + 2] = jnp.zeros((1, wi, ci), jnp.float32)

    cp_x.wait()

    @pl.when(r > 0)
    def _():
        pltpu.make_async_copy(x_hbm.at[b, pl.ds(r * hh - 1, 1), :, :],
                              sx.at[pl.ds(0, 1)], sem.at[0, 1]).wait()

    @pl.when(r < nr - 1)
    def _():
        pltpu.make_async_copy(x_hbm.at[b, pl.ds((r + 1) * hh, 1), :, :],
                              sx.at[pl.ds(hh + 1, 1)], sem.at[0, 2]).wait()

    x2 = sx[...].reshape((hh + 2) * wi, ci).astype(jnp.bfloat16)
    yd = jnp.dot(x2, wd_ref[...], preferred_element_type=jnp.float32)
    yd = (yd + bd_ref[0]).astype(jnp.bfloat16)
    row_blocks = []
    for ky in range(2):
        parts = []
        for kx in range(2):
            t = yd[:, (ky * 2 + kx) * co:(ky * 2 + kx + 1) * co]
            parts.append(t.reshape(hh + 2, wi, 1, co))
        blk = jnp.concatenate(parts, axis=2)
        row_blocks.append(blk.reshape(hh + 2, 1, wo, co))
    su[:, :, 0:co] = jnp.concatenate(row_blocks, axis=1).reshape(
        th + 4, wo, co)

    cp_c.wait()

    @pl.when(r > 0)
    def _():
        pltpu.make_async_copy(xc_hbm.at[b, pl.ds(r * th - 2, 2), :, :],
                              scf.at[pl.ds(0, 2)], sem.at[1, 1]).wait()

    @pl.when(r < nr - 1)
    def _():
        pltpu.make_async_copy(xc_hbm.at[b, pl.ds((r + 1) * th, 2), :, :],
                              scf.at[pl.ds(th + 2, 2)], sem.at[1, 2]).wait()

    su[:, :, co:2 * co] = scf[...].astype(jnp.bfloat16)

    @pl.when(r == 0)
    def _():
        su[0:2] = jnp.zeros((2, wo, 2 * co), jnp.bfloat16)

    @pl.when(r == nr - 1)
    def _():
        su[th + 2:th + 4] = jnp.zeros((2, wo, 2 * co), jnp.bfloat16)

    _shift_cols(su, -1, sl)
    _shift_cols(su, +1, sr)
    acc = _conv3x3([sl, su, sr], w1_ref, (th + 2) * wo, wo)
    y1 = jnp.maximum(acc * s1_ref[0] + b1_ref[0], 0.0)
    hb[...] = y1.astype(jnp.bfloat16).reshape(th + 2, wo, co)

    @pl.when(r == 0)
    def _():
        hb[0:1] = jnp.zeros((1, wo, co), jnp.bfloat16)

    @pl.when(r == nr - 1)
    def _():
        hb[th + 1:th + 2] = jnp.zeros((1, wo, co), jnp.bfloat16)

    _shift_cols(hb, -1, hl)
    _shift_cols(hb, +1, hr)
    acc2 = _conv3x3([hl, hb, hr], w2_ref, th * wo, wo)
    y2 = jnp.maximum(acc2 * s2_ref[0] + b2_ref[0], 0.0)
    o_ref[0] = y2.reshape(th, wo, co)


def _fold_bn(conv_b, gamma, beta, mean, var):
    scale = gamma / jnp.sqrt(var + _BN_EPS)
    bias = beta + (conv_b - mean) * scale
    return scale[None, :], bias[None, :]


def _merge_taps(w_hwio):
    return jnp.stack([jnp.concatenate([w_hwio[dy, dx] for dy in range(3)],
                                      axis=1) for dx in range(3)])


@jax.jit
def _forward(ct_w, ct_b, c1_w, c1_b, bn1_g, bn1_b, bn1_m, bn1_v,
             c2_w, c2_b, bn2_g, bn2_b, bn2_m, bn2_v, x_nchw, xcopy_nchw):
    n, ci, hi, wi = x_nchw.shape
    co = ct_b.shape[0]
    ho, wo = 2 * hi, 2 * wi
    th = ho // _NSTRIPS

    x = jnp.transpose(x_nchw, (0, 2, 3, 1))
    xc = jnp.transpose(xcopy_nchw, (0, 2, 3, 1))

    wd = jnp.transpose(ct_w, (2, 3, 0, 1)).reshape(4, ci, co)
    wd = jnp.concatenate([wd[0], wd[1], wd[2], wd[3]],
                         axis=1).astype(jnp.bfloat16)
    bd = jnp.tile(ct_b, 4)[None, :]
    w1 = _merge_taps(jnp.transpose(c1_w, (2, 3, 1, 0))).astype(jnp.bfloat16)
    s1, b1 = _fold_bn(c1_b, bn1_g, bn1_b, bn1_m, bn1_v)
    w2 = _merge_taps(jnp.transpose(c2_w, (2, 3, 1, 0))).astype(jnp.bfloat16)
    s2, b2 = _fold_bn(c2_b, bn2_g, bn2_b, bn2_m, bn2_v)

    out = pl.pallas_call(
        _fused_kernel,
        grid=(n, _NSTRIPS),
        in_specs=[
            pl.BlockSpec(memory_space=pl.ANY),
            pl.BlockSpec(memory_space=pl.ANY),
            pl.BlockSpec((ci, 4 * co), lambda b, r: (0, 0)),
            pl.BlockSpec((1, 4 * co), lambda b, r: (0, 0)),
            pl.BlockSpec((3, 2 * co, 3 * co), lambda b, r: (0, 0, 0)),
            pl.BlockSpec((1, co), lambda b, r: (0, 0)),
            pl.BlockSpec((1, co), lambda b, r: (0, 0)),
            pl.BlockSpec((3, co, 3 * co), lambda b, r: (0, 0, 0)),
            pl.BlockSpec((1, co), lambda b, r: (0, 0)),
            pl.BlockSpec((1, co), lambda b, r: (0, 0)),
        ],
        out_specs=pl.BlockSpec((1, th, wo, co), lambda b, r: (b, r, 0, 0)),
        out_shape=jax.ShapeDtypeStruct((n, ho, wo, co), jnp.float32),
        scratch_shapes=[
            pltpu.VMEM((th // 2 + 2, wi, ci), jnp.float32),
            pltpu.VMEM((th + 4, wo, co), jnp.float32),
            pltpu.VMEM((th + 4, wo, 2 * co), jnp.bfloat16),
            pltpu.VMEM((th + 4, wo, 2 * co), jnp.bfloat16),
            pltpu.VMEM((th + 4, wo, 2 * co), jnp.bfloat16),
            pltpu.VMEM((th + 2, wo, co), jnp.bfloat16),
            pltpu.VMEM((th + 2, wo, co), jnp.bfloat16),
            pltpu.VMEM((th + 2, wo, co), jnp.bfloat16),
            pltpu.SemaphoreType.DMA((2, 3)),
        ],
        compiler_params=pltpu.CompilerParams(
            dimension_semantics=("parallel", "parallel"),
            vmem_limit_bytes=_VMEM_LIMIT),
    )(x, xc, wd, bd, w1, s1, b1, w2, s2, b2)
    return jnp.transpose(out, (0, 3, 1, 2))


def kernel(ct_w, ct_b, c1_w, c1_b, bn1_g, bn1_b, bn1_m, bn1_v,
           c2_w, c2_b, bn2_g, bn2_b, bn2_m, bn2_v, x_nchw, xcopy_nchw):
    return _forward(ct_w, ct_b, c1_w, c1_b, bn1_g, bn1_b, bn1_m, bn1_v,
                    c2_w, c2_b, bn2_g, bn2_b, bn2_m, bn2_v,
                    x_nchw, xcopy_nchw)
